# baseline (speedup 1.0000x reference)
import functools

import jax
import jax.numpy as jnp
from jax import lax
from jax.experimental import pallas as pl
from jax.experimental.pallas import tpu as pltpu
from jax.experimental.pallas import tpu_sc as plsc

_VOCAB = 100000
_D = 2048
_B = 4
_S = 4096

_NC = 2
_NS = 16
_NW = _NC * _NS
_POS_PER_W = _S // _NW
_K = 16
_NCHUNK = _POS_PER_W // _K
_LANES = 16
_VECS_PER_ROW = _D // _LANES


def _emb_body(x_hbm, table_hbm, pe_hbm, out_hbm, idx_v, pe_v, rows_v, sem):
    wid = lax.axis_index("s") * _NC + lax.axis_index("c")
    s0 = wid * _POS_PER_W

    def chunk_body(j, carry):
        sbase = s0 + j * _K
        pltpu.sync_copy(pe_hbm.at[pl.ds(sbase, _K), :], pe_v)

        def batch_body(b, carry2):
            flat0 = b * _S + sbase
            pltpu.sync_copy(x_hbm.at[pl.ds(flat0, _K)], idx_v)
            pltpu.async_copy(table_hbm.at[idx_v], rows_v, sem).wait()

            def add_body(i, carry3):
                for r in range(_K):
                    v = pe_v[r, pl.ds(i * _LANES, _LANES)]
                    plsc.addupdate(rows_v.at[r, pl.ds(i * _LANES, _LANES)], v)
                return carry3

            lax.fori_loop(0, _VECS_PER_ROW, add_body, 0)
            pltpu.sync_copy(rows_v, out_hbm.at[pl.ds(flat0, _K), :])
            return carry2

        lax.fori_loop(0, _B, batch_body, 0)
        return carry

    lax.fori_loop(0, _NCHUNK, chunk_body, 0)


@jax.jit
def kernel(x, table, pe):
    x_flat = x.reshape(-1)
    emb = pl.kernel(
        _emb_body,
        out_type=jax.ShapeDtypeStruct((_B * _S, _D), jnp.float32),
        mesh=plsc.VectorSubcoreMesh(core_axis_name="c", subcore_axis_name="s"),
        scratch_types=[
            pltpu.VMEM((_K,), jnp.int32),
            pltpu.VMEM((_K, _D), jnp.float32),
            pltpu.VMEM((_K, _D), jnp.float32),
            pltpu.SemaphoreType.DMA,
        ],
    )
    out = emb(x_flat, table, pe)
    return out.reshape(_B, _S, _D)
